# baseline (speedup 1.0000x reference)
import jax
import jax.numpy as jnp
from jax.experimental import pallas as pl
from jax.experimental.pallas import tpu as pltpu
from jax.experimental.pallas import tpu_sc as plsc

B, C, L = 64, 1, 1632
H, D, K = 64, 512, 1024
T = L // 4
N = B * T
BLK = 256
GRID = N // BLK
BLK_E = 512
GRID_E = N // BLK_E


def _enc_vq_kernel(img4_ref, w1e_ref, b1e_ref, W2e_ref, b2e_ref, w3e_ref,
                   b3e_ref, cbT_ref, idx_ref, commit_ref, c2_ref):
    step = pl.program_id(0)

    @pl.when(step == 0)
    def _init():
        cbT = cbT_ref[...]
        c2_ref[...] = jnp.sum(cbT * cbT, axis=0, keepdims=True)
        commit_ref[...] = jnp.zeros_like(commit_ref)

    img4 = img4_ref[...]
    w1e = w1e_ref[...]
    b1e = b1e_ref[...]
    h1e = jax.nn.relu(img4[:, 0:1] * w1e[0:1, :] + img4[:, 1:2] * w1e[1:2, :] + b1e)
    h1o = jax.nn.relu(img4[:, 2:3] * w1e[0:1, :] + img4[:, 3:4] * w1e[1:2, :] + b1e)
    h12 = jnp.concatenate([h1e, h1o], axis=1)
    h2 = jax.nn.relu(jnp.dot(h12, W2e_ref[...], preferred_element_type=jnp.float32)
                     + b2e_ref[...])
    x = jnp.dot(h2, w3e_ref[...], preferred_element_type=jnp.float32) + b3e_ref[...]

    x2 = jnp.sum(x * x, axis=1, keepdims=True)
    scores = jnp.dot(x, cbT_ref[...], preferred_element_type=jnp.float32)
    dist = x2 - 2.0 * scores + c2_ref[...]
    m = jnp.min(dist, axis=1, keepdims=True)
    iota = jax.lax.broadcasted_iota(jnp.int32, dist.shape, 1)
    idx_ref[...] = jnp.min(jnp.where(dist == m, iota, K), axis=1, keepdims=True)
    commit_ref[...] += jnp.sum(m).reshape(1, 1)


def _table_kernel(cb_ref, w1d_ref, table_ref):
    table_ref[...] = jnp.dot(cb_ref[...], w1d_ref[...],
                             preferred_element_type=jnp.float32)


def _dec_kernel(h1pre_ref, img4_ref, b1d_ref, W2d_ref, b2d_ref, W3d_ref,
                b3d_ref, out4_ref, recon_ref):
    step = pl.program_id(0)

    @pl.when(step == 0)
    def _init():
        recon_ref[...] = jnp.zeros_like(recon_ref)

    h1d = jax.nn.relu(h1pre_ref[...] + b1d_ref[...])
    h2d = jax.nn.relu(jnp.dot(h1d, W2d_ref[...], preferred_element_type=jnp.float32)
                      + b2d_ref[...])
    out4 = jnp.dot(h2d, W3d_ref[...], preferred_element_type=jnp.float32) + b3d_ref[...]
    out4_ref[...] = out4
    diff = img4_ref[...] - out4
    recon_ref[...] += jnp.sum(diff * diff).reshape(1, 1)


def _sc_gather(table, idx2d, window):
    n = idx2d.shape[1]
    value_dim = table.shape[1]
    mesh = plsc.VectorSubcoreMesh(core_axis_name="core", subcore_axis_name="subcore")

    @pl.kernel(out_type=jax.ShapeDtypeStruct((n, value_dim), table.dtype),
               mesh=mesh)
    def kern(x_hbm, i_hbm, o_hbm):
        def body(i_vmem, o_vmem):
            pltpu.sync_copy(x_hbm.at[i_vmem.at[0]], o_vmem)

        pltpu.emit_pipeline(
            body,
            grid=(n // window,),
            in_specs=[pl.BlockSpec((1, window), index_map=lambda i: (0, i))],
            out_specs=[pl.BlockSpec((window, value_dim), index_map=lambda i: (i, 0))],
            core_axis_name=("core", "subcore"),
            dimension_semantics=(pltpu.PARALLEL,),
        )(i_hbm, o_hbm)

    return kern(table, idx2d)


@jax.jit
def kernel(img, enc_w1, enc_b1, enc_w2, enc_b2, enc_w3, enc_b3, codebook,
           dec_w1, dec_b1, dec_w2, dec_b2, dec_w3, dec_b3):
    f32 = jnp.float32
    img4 = img.reshape(B, T, 4).reshape(N, 4)

    w1e = enc_w1[:, 0, :].T
    W2e = enc_w2.transpose(2, 1, 0).reshape(2 * H, H)
    w3e = enc_w3[:, :, 0].T
    cbT = codebook.T
    w1d = dec_w1[:, :, ::-1].transpose(0, 2, 1).reshape(D, 2 * H)
    b1d = jnp.tile(dec_b1, 2)[None, :]
    dw2f = dec_w2[:, :, ::-1]
    zer = jnp.zeros((H, H), f32)
    W2d = jnp.concatenate([
        jnp.concatenate([dw2f[:, :, 0], dw2f[:, :, 1], zer, zer], axis=1),
        jnp.concatenate([zer, zer, dw2f[:, :, 0], dw2f[:, :, 1]], axis=1),
    ], axis=0)
    b2d = jnp.tile(dec_b2, 4)[None, :]
    w3v = dec_w3[0, :, 0]
    zv = jnp.zeros((H,), f32)
    W3d = jnp.stack([
        jnp.concatenate([w3v, zv, zv, zv]),
        jnp.concatenate([zv, w3v, zv, zv]),
        jnp.concatenate([zv, zv, w3v, zv]),
        jnp.concatenate([zv, zv, zv, w3v]),
    ], axis=1)
    b3d = dec_b3[None, :]

    full = lambda shape: pl.BlockSpec(shape, lambda i: (0, 0))

    idx, commit_acc = pl.pallas_call(
        _enc_vq_kernel,
        grid=(GRID,),
        in_specs=[
            pl.BlockSpec((BLK, 4), lambda i: (i, 0)),
            full((2, H)), full((1, H)), full((2 * H, H)), full((1, H)),
            full((H, D)), full((1, D)), full((D, K)),
        ],
        out_specs=[
            pl.BlockSpec((BLK, 1), lambda i: (i, 0)),
            pl.BlockSpec((1, 1), lambda i: (0, 0)),
        ],
        out_shape=[
            jax.ShapeDtypeStruct((N, 1), jnp.int32),
            jax.ShapeDtypeStruct((1, 1), f32),
        ],
        scratch_shapes=[pltpu.VMEM((1, K), f32)],
    )(img4, w1e, enc_b1[None, :], W2e, enc_b2[None, :], w3e, enc_b3[None, :], cbT)

    table = pl.pallas_call(
        _table_kernel,
        in_specs=[pl.BlockSpec((K, D), lambda: (0, 0)),
                  pl.BlockSpec((D, 2 * H), lambda: (0, 0))],
        out_specs=pl.BlockSpec((K, 2 * H), lambda: (0, 0)),
        out_shape=jax.ShapeDtypeStruct((K, 2 * H), f32),
    )(codebook, w1d)

    idx2d = idx.reshape(1, N)
    h1pre = _sc_gather(table, idx2d, window=128)
    q0 = _sc_gather(codebook[:, :D // 2], idx2d, window=128)
    q1 = _sc_gather(codebook[:, D // 2:], idx2d, window=128)

    out4, recon_acc = pl.pallas_call(
        _dec_kernel,
        grid=(GRID_E,),
        in_specs=[
            pl.BlockSpec((BLK_E, 2 * H), lambda i: (i, 0)),
            pl.BlockSpec((BLK_E, 4), lambda i: (i, 0)),
            full((1, 2 * H)), full((2 * H, 4 * H)), full((1, 4 * H)),
            full((4 * H, 4)), full((1, 1)),
        ],
        out_specs=[
            pl.BlockSpec((BLK_E, 4), lambda i: (i, 0)),
            pl.BlockSpec((1, 1), lambda i: (0, 0)),
        ],
        out_shape=[
            jax.ShapeDtypeStruct((N, 4), f32),
            jax.ShapeDtypeStruct((1, 1), f32),
        ],
    )(h1pre, img4, b1d, W2d, b2d, W3d, b3d)

    out = out4.reshape(B, T * 4)[:, None, :]
    recon_loss = recon_acc[0, 0] / (B * C * L)
    commit_loss = commit_acc[0, 0] / (B * T * D)
    indices = idx[:, 0].reshape(B, T)
    quantized = jnp.concatenate(
        [q0.reshape(B, T, D // 2).transpose(0, 2, 1),
         q1.reshape(B, T, D // 2).transpose(0, 2, 1)], axis=1)
    return (out, recon_loss, commit_loss, indices, quantized)


# speedup vs baseline: 1.4517x; 1.4517x over previous
import functools

import jax
import jax.numpy as jnp
from jax.experimental import pallas as pl
from jax.experimental.pallas import tpu as pltpu
from jax.experimental.pallas import tpu_sc as plsc

B, C, L = 64, 1, 1632
H, D, K = 64, 512, 1024
T = L // 4
N = B * T
BLK = 256
GRID = N // BLK
BLK_E = 512
GRID_E = N // BLK_E


def _enc_vq_kernel(img4_ref, w1e_ref, b1e_ref, W2e_ref, b2e_ref, w3e_ref,
                   b3e_ref, cbT_ref, idx_ref, commit_ref, c2_ref):
    step = pl.program_id(0)

    @pl.when(step == 0)
    def _init():
        cbT = cbT_ref[...]
        c2_ref[...] = jnp.sum(cbT * cbT, axis=0, keepdims=True)
        commit_ref[...] = jnp.zeros_like(commit_ref)

    img4 = img4_ref[...]
    w1e = w1e_ref[...]
    b1e = b1e_ref[...]
    h1e = jax.nn.relu(img4[:, 0:1] * w1e[0:1, :] + img4[:, 1:2] * w1e[1:2, :] + b1e)
    h1o = jax.nn.relu(img4[:, 2:3] * w1e[0:1, :] + img4[:, 3:4] * w1e[1:2, :] + b1e)
    h12 = jnp.concatenate([h1e, h1o], axis=1)
    h2 = jax.nn.relu(jnp.dot(h12, W2e_ref[...], preferred_element_type=jnp.float32)
                     + b2e_ref[...])
    x = jnp.dot(h2, w3e_ref[...], preferred_element_type=jnp.float32) + b3e_ref[...]

    x2 = jnp.sum(x * x, axis=1, keepdims=True)
    scores = jnp.dot(x, cbT_ref[...], preferred_element_type=jnp.float32)
    dist = x2 - 2.0 * scores + c2_ref[...]
    m = jnp.min(dist, axis=1, keepdims=True)
    iota = jax.lax.broadcasted_iota(jnp.int32, dist.shape, 1)
    idx_ref[...] = jnp.min(jnp.where(dist == m, iota, K), axis=1, keepdims=True)
    commit_ref[...] += jnp.sum(m).reshape(1, 1)


def _table_kernel(cb_ref, w1d_ref, table_ref):
    table_ref[...] = jnp.dot(cb_ref[...], w1d_ref[...],
                             preferred_element_type=jnp.float32)


def _dec_kernel(h1pre_ref, img4_ref, b1d_ref, W2d_ref, b2d_ref, W3d_ref,
                b3d_ref, out4_ref, recon_ref):
    step = pl.program_id(0)

    @pl.when(step == 0)
    def _init():
        recon_ref[...] = jnp.zeros_like(recon_ref)

    h1d = jax.nn.relu(h1pre_ref[...] + b1d_ref[...])
    h2d = jax.nn.relu(jnp.dot(h1d, W2d_ref[...], preferred_element_type=jnp.float32)
                      + b2d_ref[...])
    out4 = jnp.dot(h2d, W3d_ref[...], preferred_element_type=jnp.float32) + b3d_ref[...]
    out4_ref[...] = out4
    diff = img4_ref[...] - out4
    recon_ref[...] += jnp.sum(diff * diff).reshape(1, 1)


_NW = 32


def _sc_gather(table, idx, chunk):
    n = idx.shape[0]
    dcol = table.shape[1]
    per_w = n // _NW
    nchunk = per_w // chunk
    mesh = plsc.VectorSubcoreMesh(core_axis_name="c", subcore_axis_name="s")

    @functools.partial(
        pl.kernel, mesh=mesh,
        out_type=jax.ShapeDtypeStruct((n, dcol), table.dtype),
        scratch_types=[pltpu.VMEM((chunk,), jnp.int32),
                       pltpu.VMEM((chunk, dcol), table.dtype),
                       pltpu.SemaphoreType.DMA])
    def kern(table_hbm, idx_hbm, out_hbm, idx_v, rows_v, sem):
        wid = jax.lax.axis_index("s") * 2 + jax.lax.axis_index("c")
        base = wid * per_w

        @pl.loop(0, nchunk)
        def _(ci):
            off = base + ci * chunk
            pltpu.sync_copy(idx_hbm.at[pl.ds(off, chunk)], idx_v)
            pltpu.async_copy(table_hbm.at[idx_v], rows_v, sem).wait()
            pltpu.sync_copy(rows_v, out_hbm.at[pl.ds(off, chunk)])

    return kern(table, idx)


@jax.jit
def kernel(img, enc_w1, enc_b1, enc_w2, enc_b2, enc_w3, enc_b3, codebook,
           dec_w1, dec_b1, dec_w2, dec_b2, dec_w3, dec_b3):
    f32 = jnp.float32
    img4 = img.reshape(B, T, 4).reshape(N, 4)

    w1e = enc_w1[:, 0, :].T
    W2e = enc_w2.transpose(2, 1, 0).reshape(2 * H, H)
    w3e = enc_w3[:, :, 0].T
    cbT = codebook.T
    w1d = dec_w1[:, :, ::-1].transpose(0, 2, 1).reshape(D, 2 * H)
    b1d = jnp.tile(dec_b1, 2)[None, :]
    dw2f = dec_w2[:, :, ::-1]
    zer = jnp.zeros((H, H), f32)
    W2d = jnp.concatenate([
        jnp.concatenate([dw2f[:, :, 0], dw2f[:, :, 1], zer, zer], axis=1),
        jnp.concatenate([zer, zer, dw2f[:, :, 0], dw2f[:, :, 1]], axis=1),
    ], axis=0)
    b2d = jnp.tile(dec_b2, 4)[None, :]
    w3v = dec_w3[0, :, 0]
    zv = jnp.zeros((H,), f32)
    W3d = jnp.stack([
        jnp.concatenate([w3v, zv, zv, zv]),
        jnp.concatenate([zv, w3v, zv, zv]),
        jnp.concatenate([zv, zv, w3v, zv]),
        jnp.concatenate([zv, zv, zv, w3v]),
    ], axis=1)
    b3d = dec_b3[None, :]

    full = lambda shape: pl.BlockSpec(shape, lambda i: (0, 0))

    idx, commit_acc = pl.pallas_call(
        _enc_vq_kernel,
        grid=(GRID,),
        in_specs=[
            pl.BlockSpec((BLK, 4), lambda i: (i, 0)),
            full((2, H)), full((1, H)), full((2 * H, H)), full((1, H)),
            full((H, D)), full((1, D)), full((D, K)),
        ],
        out_specs=[
            pl.BlockSpec((BLK, 1), lambda i: (i, 0)),
            pl.BlockSpec((1, 1), lambda i: (0, 0)),
        ],
        out_shape=[
            jax.ShapeDtypeStruct((N, 1), jnp.int32),
            jax.ShapeDtypeStruct((1, 1), f32),
        ],
        scratch_shapes=[pltpu.VMEM((1, K), f32)],
    )(img4, w1e, enc_b1[None, :], W2e, enc_b2[None, :], w3e, enc_b3[None, :], cbT)

    table = pl.pallas_call(
        _table_kernel,
        in_specs=[pl.BlockSpec((K, D), lambda: (0, 0)),
                  pl.BlockSpec((D, 2 * H), lambda: (0, 0))],
        out_specs=pl.BlockSpec((K, 2 * H), lambda: (0, 0)),
        out_shape=jax.ShapeDtypeStruct((K, 2 * H), f32),
    )(codebook, w1d)

    idx1d = idx.reshape(N)
    h1pre = _sc_gather(table, idx1d, chunk=816)
    q = _sc_gather(codebook, idx1d, chunk=136)

    out4, recon_acc = pl.pallas_call(
        _dec_kernel,
        grid=(GRID_E,),
        in_specs=[
            pl.BlockSpec((BLK_E, 2 * H), lambda i: (i, 0)),
            pl.BlockSpec((BLK_E, 4), lambda i: (i, 0)),
            full((1, 2 * H)), full((2 * H, 4 * H)), full((1, 4 * H)),
            full((4 * H, 4)), full((1, 1)),
        ],
        out_specs=[
            pl.BlockSpec((BLK_E, 4), lambda i: (i, 0)),
            pl.BlockSpec((1, 1), lambda i: (0, 0)),
        ],
        out_shape=[
            jax.ShapeDtypeStruct((N, 4), f32),
            jax.ShapeDtypeStruct((1, 1), f32),
        ],
    )(h1pre, img4, b1d, W2d, b2d, W3d, b3d)

    out = out4.reshape(B, T * 4)[:, None, :]
    recon_loss = recon_acc[0, 0] / (B * C * L)
    commit_loss = commit_acc[0, 0] / (B * T * D)
    indices = idx[:, 0].reshape(B, T)
    quantized = q.reshape(B, T, D).transpose(0, 2, 1)
    return (out, recon_loss, commit_loss, indices, quantized)


# speedup vs baseline: 4.4594x; 3.0719x over previous
import functools

import jax
import jax.numpy as jnp
from jax.experimental import pallas as pl
from jax.experimental.pallas import tpu as pltpu
from jax.sharding import PartitionSpec as P

B, C, L = 64, 1, 1632
H, D, K = 64, 512, 1024
T = L // 4
N = B * T
BLK = T


def _vq_kernel(img4_ref, w1e_ref, b1e_ref, W2e_ref, b2e_ref, w3e_ref, b3e_ref,
               cbT_ref, cbh_ref, cbl_ref, w1d_ref, b1d_ref, W2d_ref, b2d_ref,
               W3d_ref, b3d_ref, out4_ref, idx_ref, qT_ref, commit_ref,
               recon_ref, c2_ref):
    step = pl.program_id(0)

    @pl.when(step == 0)
    def _init():
        cbT = cbT_ref[...]
        c2_ref[...] = jnp.sum(cbT * cbT, axis=0, keepdims=True)
        commit_ref[...] = jnp.zeros_like(commit_ref)
        recon_ref[...] = jnp.zeros_like(recon_ref)

    img4 = img4_ref[...]
    w1e = w1e_ref[...]
    b1e = b1e_ref[...]
    h1e = jax.nn.relu(img4[:, 0:1] * w1e[0:1, :] + img4[:, 1:2] * w1e[1:2, :] + b1e)
    h1o = jax.nn.relu(img4[:, 2:3] * w1e[0:1, :] + img4[:, 3:4] * w1e[1:2, :] + b1e)
    h12 = jnp.concatenate([h1e, h1o], axis=1)
    h2 = jax.nn.relu(jnp.dot(h12, W2e_ref[...], preferred_element_type=jnp.float32)
                     + b2e_ref[...])
    x = jnp.dot(h2, w3e_ref[...], preferred_element_type=jnp.float32) + b3e_ref[...]

    x2 = jnp.sum(x * x, axis=1, keepdims=True)
    scores = jnp.dot(x, cbT_ref[...], preferred_element_type=jnp.float32)
    dist = x2 - 2.0 * scores + c2_ref[...]
    m = jnp.min(dist, axis=1, keepdims=True)
    iota = jax.lax.broadcasted_iota(jnp.int32, dist.shape, 1)
    idx = jnp.min(jnp.where(dist == m, iota, K), axis=1, keepdims=True)
    idx_ref[...] = idx

    onehot = (iota == idx).astype(jnp.bfloat16)
    q = (jnp.dot(onehot, cbh_ref[...], preferred_element_type=jnp.float32)
         + jnp.dot(onehot, cbl_ref[...], preferred_element_type=jnp.float32))
    qT_ref[0] = q.T

    h1d = jax.nn.relu(jnp.dot(q, w1d_ref[...], preferred_element_type=jnp.float32)
                      + b1d_ref[...])
    h2d = jax.nn.relu(jnp.dot(h1d, W2d_ref[...], preferred_element_type=jnp.float32)
                      + b2d_ref[...])
    out4 = jnp.dot(h2d, W3d_ref[...], preferred_element_type=jnp.float32) + b3d_ref[...]
    out4_ref[...] = out4

    commit_ref[...] += jnp.sum(m).reshape(1, 1)
    diff = img4 - out4
    recon_ref[...] += jnp.sum(diff * diff).reshape(1, 1)


def _run_shard(img4, w1e, b1e, W2e, b2e, w3e, b3e, cbT, cbh, cbl, w1d, b1d,
               W2d, b2d, W3d, b3d):
    nb = img4.shape[0] // BLK
    f32 = jnp.float32
    full = lambda shape: pl.BlockSpec(shape, lambda i: tuple(0 for _ in shape))
    return pl.pallas_call(
        _vq_kernel,
        grid=(nb,),
        in_specs=[
            pl.BlockSpec((BLK, 4), lambda i: (i, 0)),
            full((2, H)), full((1, H)), full((2 * H, H)), full((1, H)),
            full((H, D)), full((1, D)), full((D, K)), full((K, D)),
            full((K, D)),
            full((D, 2 * H)), full((1, 2 * H)), full((2 * H, 4 * H)),
            full((1, 4 * H)), full((4 * H, 4)), full((1, 1)),
        ],
        out_specs=[
            pl.BlockSpec((BLK, 4), lambda i: (i, 0)),
            pl.BlockSpec((BLK, 1), lambda i: (i, 0)),
            pl.BlockSpec((1, D, BLK), lambda i: (i, 0, 0)),
            pl.BlockSpec((1, 1), lambda i: (0, 0)),
            pl.BlockSpec((1, 1), lambda i: (0, 0)),
        ],
        out_shape=[
            jax.ShapeDtypeStruct((nb * BLK, 4), f32),
            jax.ShapeDtypeStruct((nb * BLK, 1), jnp.int32),
            jax.ShapeDtypeStruct((nb, D, BLK), f32),
            jax.ShapeDtypeStruct((1, 1), f32),
            jax.ShapeDtypeStruct((1, 1), f32),
        ],
        scratch_shapes=[pltpu.VMEM((1, K), f32)],
    )(img4, w1e, b1e, W2e, b2e, w3e, b3e, cbT, cbh, cbl, w1d, b1d, W2d, b2d,
      W3d, b3d)


@jax.jit
def kernel(img, enc_w1, enc_b1, enc_w2, enc_b2, enc_w3, enc_b3, codebook,
           dec_w1, dec_b1, dec_w2, dec_b2, dec_w3, dec_b3):
    f32 = jnp.float32
    img4 = img.reshape(B, T, 4).reshape(N, 4)

    w1e = enc_w1[:, 0, :].T
    W2e = enc_w2.transpose(2, 1, 0).reshape(2 * H, H)
    w3e = enc_w3[:, :, 0].T
    cbT = codebook.T
    cbh = codebook.astype(jnp.bfloat16)
    cbl = (codebook - cbh.astype(f32)).astype(jnp.bfloat16)
    w1d = dec_w1[:, :, ::-1].transpose(0, 2, 1).reshape(D, 2 * H)
    b1d = jnp.tile(dec_b1, 2)[None, :]
    dw2f = dec_w2[:, :, ::-1]
    zer = jnp.zeros((H, H), f32)
    W2d = jnp.concatenate([
        jnp.concatenate([dw2f[:, :, 0], dw2f[:, :, 1], zer, zer], axis=1),
        jnp.concatenate([zer, zer, dw2f[:, :, 0], dw2f[:, :, 1]], axis=1),
    ], axis=0)
    b2d = jnp.tile(dec_b2, 4)[None, :]
    w3v = dec_w3[0, :, 0]
    zv = jnp.zeros((H,), f32)
    W3d = jnp.stack([
        jnp.concatenate([w3v, zv, zv, zv]),
        jnp.concatenate([zv, w3v, zv, zv]),
        jnp.concatenate([zv, zv, w3v, zv]),
        jnp.concatenate([zv, zv, zv, w3v]),
    ], axis=1)
    b3d = dec_b3[None, :]

    wargs = (w1e, enc_b1[None, :], W2e, enc_b2[None, :], w3e, enc_b3[None, :],
             cbT, cbh, cbl, w1d, b1d, W2d, b2d, W3d, b3d)

    devs = jax.devices()
    if len(devs) >= 2:
        mesh = jax.sharding.Mesh(devs[:2], ("b",))
        wspecs = tuple(P() for _ in wargs)
        out4, idx, qT, commit_acc, recon_acc = jax.shard_map(
            _run_shard, mesh=mesh,
            in_specs=(P("b"),) + wspecs,
            out_specs=(P("b"), P("b"), P("b"), P("b"), P("b")),
            check_vma=False,
        )(img4, *wargs)
        commit_sum = jnp.sum(commit_acc)
        recon_sum = jnp.sum(recon_acc)
    else:
        out4, idx, qT, commit_acc, recon_acc = _run_shard(img4, *wargs)
        commit_sum = commit_acc[0, 0]
        recon_sum = recon_acc[0, 0]

    out = out4.reshape(B, T * 4)[:, None, :]
    recon_loss = recon_sum / (B * C * L)
    commit_loss = commit_sum / (B * T * D)
    indices = idx[:, 0].reshape(B, T)
    quantized = qT
    return (out, recon_loss, commit_loss, indices, quantized)


# speedup vs baseline: 10.2951x; 2.3086x over previous
import functools

import jax
import jax.numpy as jnp
from jax.experimental import pallas as pl
from jax.experimental.pallas import tpu as pltpu
from jax.sharding import PartitionSpec as P

B, C, L = 64, 1, 1632
H, D, K = 64, 512, 1024
T = L // 4
N = B * T
BLK = T


def _vq_kernel(img4_ref, w1e_ref, b1e_ref, W2e_ref, b2e_ref, w3e_ref, b3e_ref,
               cbT_ref, cbh_ref, cbl_ref, w1d_ref, b1d_ref, W2d_ref, b2d_ref,
               W3d_ref, b3d_ref, out4_ref, idx_ref, qT_ref, commit_ref,
               recon_ref, c2_ref):
    step = pl.program_id(0)

    @pl.when(step == 0)
    def _init():
        cbT = cbT_ref[...]
        c2_ref[...] = jnp.sum(cbT * cbT, axis=0, keepdims=True)
        commit_ref[...] = jnp.zeros_like(commit_ref)
        recon_ref[...] = jnp.zeros_like(recon_ref)

    img4 = img4_ref[...]
    w1e = w1e_ref[...]
    b1e = b1e_ref[...]
    h1e = jax.nn.relu(img4[:, 0:1] * w1e[0:1, :] + img4[:, 1:2] * w1e[1:2, :] + b1e)
    h1o = jax.nn.relu(img4[:, 2:3] * w1e[0:1, :] + img4[:, 3:4] * w1e[1:2, :] + b1e)
    h12 = jnp.concatenate([h1e, h1o], axis=1)
    h2 = jax.nn.relu(jnp.dot(h12, W2e_ref[...], preferred_element_type=jnp.float32)
                     + b2e_ref[...])
    x = jnp.dot(h2, w3e_ref[...], preferred_element_type=jnp.float32) + b3e_ref[...]

    x2 = jnp.sum(x * x, axis=1, keepdims=True)
    scores = jnp.dot(x, cbT_ref[...], preferred_element_type=jnp.float32)
    dist = x2 - 2.0 * scores + c2_ref[...]
    m = jnp.min(dist, axis=1, keepdims=True)
    iota = jax.lax.broadcasted_iota(jnp.int32, dist.shape, 1)
    idx = jnp.min(jnp.where(dist == m, iota, K), axis=1, keepdims=True)
    idx_ref[...] = idx

    onehot = (iota == idx).astype(jnp.bfloat16)
    q = (jnp.dot(onehot, cbh_ref[...], preferred_element_type=jnp.float32)
         + jnp.dot(onehot, cbl_ref[...], preferred_element_type=jnp.float32))
    qT_ref[0] = q.T

    h1d = jax.nn.relu(jnp.dot(q, w1d_ref[...], preferred_element_type=jnp.float32)
                      + b1d_ref[...])
    h2d = jax.nn.relu(jnp.dot(h1d, W2d_ref[...], preferred_element_type=jnp.float32)
                      + b2d_ref[...])
    out4 = jnp.dot(h2d, W3d_ref[...], preferred_element_type=jnp.float32) + b3d_ref[...]
    out4_ref[...] = out4

    commit_ref[...] += jnp.sum(m).reshape(1, 1)
    diff = img4 - out4
    recon_ref[...] += jnp.sum(diff * diff).reshape(1, 1)


def _run_shard(img4, w1e, b1e, W2e, b2e, w3e, b3e, cbT, cbh, cbl, w1d, b1d,
               W2d, b2d, W3d, b3d):
    nb = img4.shape[0] // BLK
    f32 = jnp.float32
    full = lambda shape: pl.BlockSpec(shape, lambda i: tuple(0 for _ in shape))
    return pl.pallas_call(
        _vq_kernel,
        grid=(nb,),
        in_specs=[
            pl.BlockSpec((BLK, 4), lambda i: (i, 0)),
            full((2, H)), full((1, H)), full((2 * H, H)), full((1, H)),
            full((H, D)), full((1, D)), full((D, K)), full((K, D)),
            full((K, D)),
            full((D, 2 * H)), full((1, 2 * H)), full((2 * H, 4 * H)),
            full((1, 4 * H)), full((4 * H, 4)), full((1, 1)),
        ],
        out_specs=[
            pl.BlockSpec((BLK, 4), lambda i: (i, 0)),
            pl.BlockSpec((BLK, 1), lambda i: (i, 0)),
            pl.BlockSpec((1, D, BLK), lambda i: (i, 0, 0)),
            pl.BlockSpec((1, 1), lambda i: (0, 0)),
            pl.BlockSpec((1, 1), lambda i: (0, 0)),
        ],
        out_shape=[
            jax.ShapeDtypeStruct((nb * BLK, 4), f32),
            jax.ShapeDtypeStruct((nb * BLK, 1), jnp.int32),
            jax.ShapeDtypeStruct((nb, D, BLK), f32),
            jax.ShapeDtypeStruct((1, 1), f32),
            jax.ShapeDtypeStruct((1, 1), f32),
        ],
        scratch_shapes=[pltpu.VMEM((1, K), f32)],
    )(img4, w1e, b1e, W2e, b2e, w3e, b3e, cbT, cbh, cbl, w1d, b1d, W2d, b2d,
      W3d, b3d)


@jax.jit
def kernel(img, enc_w1, enc_b1, enc_w2, enc_b2, enc_w3, enc_b3, codebook,
           dec_w1, dec_b1, dec_w2, dec_b2, dec_w3, dec_b3):
    f32 = jnp.float32
    img4 = img.reshape(B, T, 4).reshape(N, 4)

    w1e = enc_w1[:, 0, :].T
    W2e = enc_w2.transpose(2, 1, 0).reshape(2 * H, H)
    w3e = enc_w3[:, :, 0].T
    cbT = codebook.T
    cbh = codebook.astype(jnp.bfloat16)
    cbl = (codebook - cbh.astype(f32)).astype(jnp.bfloat16)
    w1d = dec_w1[:, :, ::-1].transpose(0, 2, 1).reshape(D, 2 * H)
    b1d = jnp.tile(dec_b1, 2)[None, :]
    dw2f = dec_w2[:, :, ::-1]
    zer = jnp.zeros((H, H), f32)
    W2d = jnp.concatenate([
        jnp.concatenate([dw2f[:, :, 0], dw2f[:, :, 1], zer, zer], axis=1),
        jnp.concatenate([zer, zer, dw2f[:, :, 0], dw2f[:, :, 1]], axis=1),
    ], axis=0)
    b2d = jnp.tile(dec_b2, 4)[None, :]
    w3v = dec_w3[0, :, 0]
    zv = jnp.zeros((H,), f32)
    W3d = jnp.stack([
        jnp.concatenate([w3v, zv, zv, zv]),
        jnp.concatenate([zv, w3v, zv, zv]),
        jnp.concatenate([zv, zv, w3v, zv]),
        jnp.concatenate([zv, zv, zv, w3v]),
    ], axis=1)
    b3d = dec_b3[None, :]

    wargs = (w1e, enc_b1[None, :], W2e, enc_b2[None, :], w3e, enc_b3[None, :],
             cbT, cbh, cbl, w1d, b1d, W2d, b2d, W3d, b3d)

    devs = jax.devices()
    if len(devs) >= 99:
        mesh = jax.sharding.Mesh(devs[:2], ("b",))
        wspecs = tuple(P() for _ in wargs)
        out4, idx, qT, commit_acc, recon_acc = jax.shard_map(
            _run_shard, mesh=mesh,
            in_specs=(P("b"),) + wspecs,
            out_specs=(P("b"), P("b"), P("b"), P("b"), P("b")),
            check_vma=False,
        )(img4, *wargs)
        commit_sum = jnp.sum(commit_acc)
        recon_sum = jnp.sum(recon_acc)
    else:
        out4, idx, qT, commit_acc, recon_acc = _run_shard(img4, *wargs)
        commit_sum = commit_acc[0, 0]
        recon_sum = recon_acc[0, 0]

    out = out4.reshape(B, T * 4)[:, None, :]
    recon_loss = recon_sum / (B * C * L)
    commit_loss = commit_sum / (B * T * D)
    indices = idx[:, 0].reshape(B, T)
    quantized = qT
    return (out, recon_loss, commit_loss, indices, quantized)
